# Initial kernel scaffold; baseline (speedup 1.0000x reference)
import jax
import jax.numpy as jnp
from jax.experimental import pallas as pl


def kernel(x, edge_index, edge_attr, batch, W1_0, b1_0, W2_0, b2_0, W1_1, b1_1, W2_1, b2_1):
    raise NotImplementedError("write your pallas kernel here")


# speedup vs baseline: 4.8756x; 4.8756x over previous
import functools

import jax
import jax.numpy as jnp
from jax import lax
from jax.experimental import pallas as pl
from jax.experimental.pallas import tpu as pltpu
from jax.experimental.pallas import tpu_sc as plsc

N, E, D, H, O, G = 10000, 320000, 128, 128, 128, 64
NC, NS, L = 2, 16, 16
EPW = E // (NC * NS)
C = 80
NCH = EPW // C
ZR = 80

BN = 1000
NB = N // BN


def _edge_aggr_body(h_hbm, ei_hbm, out_hbm, sidx, didx, rows, zbuf, acc, sem):
    cid = lax.axis_index("c")
    sid = lax.axis_index("s")

    zv = jnp.zeros((L,), jnp.float32)

    def zb(i, _):
        zbuf[i // (D // L), pl.ds((i % (D // L)) * L, L)] = zv
        return 0

    lax.fori_loop(0, ZR * (D // L), zb, 0)

    nz = jnp.where(sid < NS - 1, 8, 5)
    zbase = sid * 640

    def zc(j, _):
        pltpu.sync_copy(zbuf, acc.at[pl.ds(zbase + j * ZR, ZR)])
        return 0

    lax.fori_loop(0, nz, zc, 0)
    plsc.subcore_barrier()

    ebase = (cid * NS + sid) * EPW

    def body(j, _):
        off = ebase + j * C
        pltpu.sync_copy(ei_hbm.at[pl.ds(off, C)], sidx)
        pltpu.sync_copy(ei_hbm.at[pl.ds(E + off, C)], didx)
        pltpu.async_copy(h_hbm.at[sidx], rows, sem).wait()
        pltpu.sync_copy(rows, acc.at[didx], add=True)
        return 0

    lax.fori_loop(0, NCH, body, 0)
    plsc.subcore_barrier()

    def dc(j, _):
        b = zbase + j * ZR
        pltpu.sync_copy(acc.at[pl.ds(b, ZR)], out_hbm.at[cid, pl.ds(b, ZR)])
        return 0

    lax.fori_loop(0, nz, dc, 0)


_edge_aggr = functools.partial(
    pl.kernel,
    out_type=jax.ShapeDtypeStruct((NC, N, D), jnp.float32),
    mesh=plsc.VectorSubcoreMesh(core_axis_name="c", subcore_axis_name="s"),
    scratch_types=[
        pltpu.VMEM((C,), jnp.int32),
        pltpu.VMEM((C,), jnp.int32),
        pltpu.VMEM((C, D), jnp.float32),
        pltpu.VMEM((ZR, D), jnp.float32),
        pltpu.VMEM_SHARED((N, D), jnp.float32),
        pltpu.SemaphoreType.DMA,
    ],
)(_edge_aggr_body)


def _mlp1_body(x_ref, p_ref, w1_ref, b1_ref, w2_ref, b2_ref, o_ref):
    m = x_ref[...] + p_ref[0] + p_ref[1]
    t = jnp.dot(m, w1_ref[...], preferred_element_type=jnp.float32) + b1_ref[...]
    t = jnp.maximum(t, 0.0)
    y = jnp.dot(t, w2_ref[...], preferred_element_type=jnp.float32) + b2_ref[...]
    o_ref[...] = jnp.maximum(y, 0.0)


def _mlp1(x, p, w1, b1, w2, b2):
    return pl.pallas_call(
        _mlp1_body,
        grid=(NB,),
        in_specs=[
            pl.BlockSpec((BN, D), lambda i: (i, 0)),
            pl.BlockSpec((NC, BN, D), lambda i: (0, i, 0)),
            pl.BlockSpec((D, H), lambda i: (0, 0)),
            pl.BlockSpec((1, H), lambda i: (0, 0)),
            pl.BlockSpec((H, H), lambda i: (0, 0)),
            pl.BlockSpec((1, H), lambda i: (0, 0)),
        ],
        out_specs=pl.BlockSpec((BN, H), lambda i: (i, 0)),
        out_shape=jax.ShapeDtypeStruct((N, H), jnp.float32),
    )(x, p, w1, b1, w2, b2)


def _mlp2_pool_body(h_ref, q_ref, bt_ref, w1_ref, b1_ref, w2_ref, b2_ref, o_ref):
    m = h_ref[...] + q_ref[0] + q_ref[1]
    t = jnp.dot(m, w1_ref[...], preferred_element_type=jnp.float32) + b1_ref[...]
    t = jnp.maximum(t, 0.0)
    y = jnp.dot(t, w2_ref[...], preferred_element_type=jnp.float32) + b2_ref[...]
    bt = bt_ref[...].reshape(1, BN)
    onehot = (bt == lax.broadcasted_iota(jnp.int32, (G, BN), 0)).astype(jnp.float32)
    contrib = jnp.dot(onehot, y, preferred_element_type=jnp.float32)

    @pl.when(pl.program_id(0) == 0)
    def _():
        o_ref[...] = jnp.zeros_like(o_ref)

    o_ref[...] += contrib


def _mlp2_pool(h, q, bt3, w1, b1, w2, b2):
    return pl.pallas_call(
        _mlp2_pool_body,
        grid=(NB,),
        in_specs=[
            pl.BlockSpec((BN, H), lambda i: (i, 0)),
            pl.BlockSpec((NC, BN, H), lambda i: (0, i, 0)),
            pl.BlockSpec((1, 1, BN), lambda i: (i, 0, 0)),
            pl.BlockSpec((H, O), lambda i: (0, 0)),
            pl.BlockSpec((1, O), lambda i: (0, 0)),
            pl.BlockSpec((O, O), lambda i: (0, 0)),
            pl.BlockSpec((1, O), lambda i: (0, 0)),
        ],
        out_specs=pl.BlockSpec((G, O), lambda i: (0, 0)),
        out_shape=jax.ShapeDtypeStruct((G, O), jnp.float32),
    )(h, q, bt3, w1, b1, w2, b2)


def kernel(x, edge_index, edge_attr, batch, W1_0, b1_0, W2_0, b2_0,
           W1_1, b1_1, W2_1, b2_1):
    del edge_attr
    ei = edge_index.reshape(2 * E)
    p = _edge_aggr(x, ei)
    h1 = _mlp1(x, p, W1_0, b1_0[None], W2_0, b2_0[None])
    q = _edge_aggr(h1, ei)
    bt3 = batch.reshape(NB, 1, BN)
    return _mlp2_pool(h1, q, bt3, W1_1, b1_1[None], W2_1, b2_1[None])


# speedup vs baseline: 11.3483x; 2.3276x over previous
import functools

import jax
import jax.numpy as jnp
from jax import lax
from jax.experimental import pallas as pl
from jax.experimental.pallas import tpu as pltpu
from jax.experimental.pallas import tpu_sc as plsc

N, E, D, H, O, G = 10000, 320000, 128, 128, 128, 64
NC, NS, L = 2, 16, 16
EPW = E // (NC * NS)
C = 80
NCH = EPW // C
NBUF = 3
NW = NCH // NBUF
NTL = NCH - NBUF * NW
ZR = 16
DR = 80

BN = 1000
NB = N // BN


def _edge_aggr_body(h_hbm, ei_hbm, out_hbm, sidx_all,
                    didx0, didx1, didx2,
                    rows0, rows1, rows2, zbuf, acc,
                    sg0, sg1, sg2, ss0, ss1, ss2):
    didx = (didx0, didx1, didx2)
    rows = (rows0, rows1, rows2)
    sg = (sg0, sg1, sg2)
    ss = (ss0, ss1, ss2)
    cid = lax.axis_index("c")
    sid = lax.axis_index("s")

    zv = jnp.zeros((L,), jnp.float32)

    def zb(i, _):
        zbuf[i // (D // L), pl.ds((i % (D // L)) * L, L)] = zv
        return 0

    lax.fori_loop(0, ZR * (D // L), zb, 0)

    nz = jnp.where(sid < NS - 1, 640 // ZR, 400 // ZR)
    zbase = sid * 640

    def zc(j, _):
        pltpu.sync_copy(zbuf, acc.at[pl.ds(zbase + j * ZR, ZR)])
        return 0

    lax.fori_loop(0, nz, zc, 0)

    ebase = (cid * NS + sid) * EPW
    pltpu.sync_copy(ei_hbm.at[pl.ds(ebase, EPW)], sidx_all)

    plsc.subcore_barrier()

    def fire_in(cn, b):
        pltpu.async_copy(ei_hbm.at[pl.ds(E + ebase + cn * C, C)], didx[b], sg[b])
        pltpu.async_copy(h_hbm.at[sidx_all.at[pl.ds(cn * C, C)]], rows[b], sg[b])

    def wait_in(cn, b):
        pltpu.make_async_copy(
            ei_hbm.at[pl.ds(E + ebase + cn * C, C)], didx[b], sg[b]).wait()
        pltpu.make_async_copy(
            h_hbm.at[sidx_all.at[pl.ds(cn * C, C)]], rows[b], sg[b]).wait()

    def fire_s(b):
        pltpu.async_copy(rows[b], acc.at[didx[b]], ss[b], add=True)

    def wait_s(b):
        pltpu.make_async_copy(rows[b], acc.at[didx[b]], ss[b]).wait()

    for b in range(NBUF):
        fire_in(b, b)

    def body(i, _):
        for b in range(NBUF):
            j = NBUF * i + b
            wait_in(j, b)
            fire_s(b)
        for b in range(NBUF):
            jn = NBUF * (i + 1) + b

            @pl.when(jn < NCH)
            def _():
                wait_s(b)
                fire_in(jn, b)
            _ = None
        return 0

    lax.fori_loop(0, NW, body, 0)
    for b in range(NTL):
        wait_in(NBUF * NW + b, b)
        fire_s(b)
    for b in range(NTL, NBUF):
        wait_s(b)
    for b in range(NTL):
        wait_s(b)

    plsc.subcore_barrier()

    nd = jnp.where(sid < NS - 1, 640 // DR, 400 // DR)

    def dc(j, _):
        b = zbase + j * DR
        pltpu.sync_copy(acc.at[pl.ds(b, DR)], out_hbm.at[cid, pl.ds(b, DR)])
        return 0

    lax.fori_loop(0, nd, dc, 0)


_edge_aggr = functools.partial(
    pl.kernel,
    out_type=jax.ShapeDtypeStruct((NC, N, D), jnp.float32),
    mesh=plsc.VectorSubcoreMesh(core_axis_name="c", subcore_axis_name="s"),
    scratch_types=(
        [pltpu.VMEM((EPW,), jnp.int32)]
        + [pltpu.VMEM((C,), jnp.int32) for _ in range(NBUF)]
        + [pltpu.VMEM((C, D), jnp.float32) for _ in range(NBUF)]
        + [pltpu.VMEM((ZR, D), jnp.float32),
           pltpu.VMEM_SHARED((N, D), jnp.float32)]
        + [pltpu.SemaphoreType.DMA for _ in range(2 * NBUF)]
    ),
)(_edge_aggr_body)


def _mlp1_body(x_ref, p_ref, w1_ref, b1_ref, w2_ref, b2_ref, o_ref):
    m = x_ref[...] + p_ref[0] + p_ref[1]
    t = jnp.dot(m, w1_ref[...], preferred_element_type=jnp.float32) + b1_ref[...]
    t = jnp.maximum(t, 0.0)
    y = jnp.dot(t, w2_ref[...], preferred_element_type=jnp.float32) + b2_ref[...]
    o_ref[...] = jnp.maximum(y, 0.0)


def _mlp1(x, p, w1, b1, w2, b2):
    return pl.pallas_call(
        _mlp1_body,
        grid=(NB,),
        in_specs=[
            pl.BlockSpec((BN, D), lambda i: (i, 0)),
            pl.BlockSpec((NC, BN, D), lambda i: (0, i, 0)),
            pl.BlockSpec((D, H), lambda i: (0, 0)),
            pl.BlockSpec((1, H), lambda i: (0, 0)),
            pl.BlockSpec((H, H), lambda i: (0, 0)),
            pl.BlockSpec((1, H), lambda i: (0, 0)),
        ],
        out_specs=pl.BlockSpec((BN, H), lambda i: (i, 0)),
        out_shape=jax.ShapeDtypeStruct((N, H), jnp.float32),
    )(x, p, w1, b1, w2, b2)


def _mlp2_pool_body(h_ref, q_ref, bt_ref, w1_ref, b1_ref, w2_ref, b2_ref, o_ref):
    m = h_ref[...] + q_ref[0] + q_ref[1]
    t = jnp.dot(m, w1_ref[...], preferred_element_type=jnp.float32) + b1_ref[...]
    t = jnp.maximum(t, 0.0)
    y = jnp.dot(t, w2_ref[...], preferred_element_type=jnp.float32) + b2_ref[...]
    bt = bt_ref[...].reshape(1, BN)
    onehot = (bt == lax.broadcasted_iota(jnp.int32, (G, BN), 0)).astype(jnp.float32)
    contrib = jnp.dot(onehot, y, preferred_element_type=jnp.float32)

    @pl.when(pl.program_id(0) == 0)
    def _():
        o_ref[...] = jnp.zeros_like(o_ref)

    o_ref[...] += contrib


def _mlp2_pool(h, q, bt3, w1, b1, w2, b2):
    return pl.pallas_call(
        _mlp2_pool_body,
        grid=(NB,),
        in_specs=[
            pl.BlockSpec((BN, H), lambda i: (i, 0)),
            pl.BlockSpec((NC, BN, H), lambda i: (0, i, 0)),
            pl.BlockSpec((1, 1, BN), lambda i: (i, 0, 0)),
            pl.BlockSpec((H, O), lambda i: (0, 0)),
            pl.BlockSpec((1, O), lambda i: (0, 0)),
            pl.BlockSpec((O, O), lambda i: (0, 0)),
            pl.BlockSpec((1, O), lambda i: (0, 0)),
        ],
        out_specs=pl.BlockSpec((G, O), lambda i: (0, 0)),
        out_shape=jax.ShapeDtypeStruct((G, O), jnp.float32),
    )(h, q, bt3, w1, b1, w2, b2)


def kernel(x, edge_index, edge_attr, batch, W1_0, b1_0, W2_0, b2_0,
           W1_1, b1_1, W2_1, b2_1):
    del edge_attr
    ei = edge_index.reshape(2 * E)
    p = _edge_aggr(x, ei)
    h1 = _mlp1(x, p, W1_0, b1_0[None], W2_0, b2_0[None])
    q = _edge_aggr(h1, ei)
    bt3 = batch.reshape(NB, 1, BN)
    return _mlp2_pool(h1, q, bt3, W1_1, b1_1[None], W2_1, b2_1[None])


# speedup vs baseline: 12.2822x; 1.0823x over previous
import functools

import jax
import jax.numpy as jnp
from jax import lax
from jax.experimental import pallas as pl
from jax.experimental.pallas import tpu as pltpu
from jax.experimental.pallas import tpu_sc as plsc

N, E, D, H, O, G = 10000, 320000, 128, 128, 128, 64
NC, NS, L = 2, 16, 16
EPW = E // (NC * NS)
C = 40
NCH = EPW // C
NBUF = 6
NW = NCH // NBUF
NTL = NCH - NBUF * NW
ZR = 16
DR = 80

BN = 1000
NB = N // BN


def _edge_aggr_body(h_hbm, ei_hbm, out_hbm, sidx_all, *scr):
    didx = scr[0:NBUF]
    rows = scr[NBUF:2 * NBUF]
    zbuf = scr[2 * NBUF]
    acc = scr[2 * NBUF + 1]
    sg = scr[2 * NBUF + 2:3 * NBUF + 2]
    ss = scr[3 * NBUF + 2:4 * NBUF + 2]
    cid = lax.axis_index("c")
    sid = lax.axis_index("s")

    ebase = (cid * NS + sid) * EPW
    pltpu.async_copy(ei_hbm.at[pl.ds(ebase, EPW)], sidx_all, ss[0])

    zv = jnp.zeros((L,), jnp.float32)

    def zb(i, _):
        zbuf[i // (D // L), pl.ds((i % (D // L)) * L, L)] = zv
        return 0

    lax.fori_loop(0, ZR * (D // L), zb, 0)

    nz = jnp.where(sid < NS - 1, 640 // ZR, 400 // ZR)
    zbase = sid * 640

    def zc(j, _):
        pltpu.sync_copy(zbuf, acc.at[pl.ds(zbase + j * ZR, ZR)])
        return 0

    lax.fori_loop(0, nz, zc, 0)

    pltpu.make_async_copy(ei_hbm.at[pl.ds(ebase, EPW)], sidx_all, ss[0]).wait()
    plsc.subcore_barrier()

    def fire_in(cn, b):
        pltpu.async_copy(ei_hbm.at[pl.ds(E + ebase + cn * C, C)], didx[b], sg[b])
        pltpu.async_copy(h_hbm.at[sidx_all.at[pl.ds(cn * C, C)]], rows[b], sg[b])

    def wait_in(cn, b):
        pltpu.make_async_copy(
            ei_hbm.at[pl.ds(E + ebase + cn * C, C)], didx[b], sg[b]).wait()
        pltpu.make_async_copy(
            h_hbm.at[sidx_all.at[pl.ds(cn * C, C)]], rows[b], sg[b]).wait()

    def fire_s(b):
        pltpu.async_copy(rows[b], acc.at[didx[b]], ss[b], add=True)

    def wait_s(b):
        pltpu.make_async_copy(rows[b], acc.at[didx[b]], ss[b]).wait()

    for b in range(NBUF):
        fire_in(b, b)

    def body(i, _):
        for b in range(NBUF):
            j = NBUF * i + b
            wait_in(j, b)
            fire_s(b)
        for b in range(NBUF):
            jn = NBUF * (i + 1) + b

            @pl.when(jn < NCH)
            def _():
                wait_s(b)
                fire_in(jn, b)
            _ = None
        return 0

    lax.fori_loop(0, NW, body, 0)
    for b in range(NTL):
        wait_in(NBUF * NW + b, b)
        fire_s(b)
    for b in range(NTL, NBUF):
        wait_s(b)
    for b in range(NTL):
        wait_s(b)

    plsc.subcore_barrier()

    nd = jnp.where(sid < NS - 1, 640 // DR, 400 // DR)

    def dc(j, _):
        b = zbase + j * DR
        pltpu.sync_copy(acc.at[pl.ds(b, DR)], out_hbm.at[cid, pl.ds(b, DR)])
        return 0

    lax.fori_loop(0, nd, dc, 0)


_edge_aggr = functools.partial(
    pl.kernel,
    out_type=jax.ShapeDtypeStruct((NC, N, D), jnp.float32),
    mesh=plsc.VectorSubcoreMesh(core_axis_name="c", subcore_axis_name="s"),
    scratch_types=(
        [pltpu.VMEM((EPW,), jnp.int32)]
        + [pltpu.VMEM((C,), jnp.int32) for _ in range(NBUF)]
        + [pltpu.VMEM((C, D), jnp.float32) for _ in range(NBUF)]
        + [pltpu.VMEM((ZR, D), jnp.float32),
           pltpu.VMEM_SHARED((N, D), jnp.float32)]
        + [pltpu.SemaphoreType.DMA for _ in range(2 * NBUF)]
    ),
)(_edge_aggr_body)


def _mlp1_body(x_ref, p_ref, w1_ref, b1_ref, w2_ref, b2_ref, o_ref):
    m = x_ref[...] + p_ref[0] + p_ref[1]
    t = jnp.dot(m, w1_ref[...], preferred_element_type=jnp.float32) + b1_ref[...]
    t = jnp.maximum(t, 0.0)
    y = jnp.dot(t, w2_ref[...], preferred_element_type=jnp.float32) + b2_ref[...]
    o_ref[...] = jnp.maximum(y, 0.0)


def _mlp1(x, p, w1, b1, w2, b2):
    return pl.pallas_call(
        _mlp1_body,
        grid=(NB,),
        in_specs=[
            pl.BlockSpec((BN, D), lambda i: (i, 0)),
            pl.BlockSpec((NC, BN, D), lambda i: (0, i, 0)),
            pl.BlockSpec((D, H), lambda i: (0, 0)),
            pl.BlockSpec((1, H), lambda i: (0, 0)),
            pl.BlockSpec((H, H), lambda i: (0, 0)),
            pl.BlockSpec((1, H), lambda i: (0, 0)),
        ],
        out_specs=pl.BlockSpec((BN, H), lambda i: (i, 0)),
        out_shape=jax.ShapeDtypeStruct((N, H), jnp.float32),
    )(x, p, w1, b1, w2, b2)


def _mlp2_pool_body(h_ref, q_ref, bt_ref, w1_ref, b1_ref, w2_ref, b2_ref, o_ref):
    m = h_ref[...] + q_ref[0] + q_ref[1]
    t = jnp.dot(m, w1_ref[...], preferred_element_type=jnp.float32) + b1_ref[...]
    t = jnp.maximum(t, 0.0)
    y = jnp.dot(t, w2_ref[...], preferred_element_type=jnp.float32) + b2_ref[...]
    bt = bt_ref[...].reshape(1, BN)
    onehot = (bt == lax.broadcasted_iota(jnp.int32, (G, BN), 0)).astype(jnp.float32)
    contrib = jnp.dot(onehot, y, preferred_element_type=jnp.float32)

    @pl.when(pl.program_id(0) == 0)
    def _():
        o_ref[...] = jnp.zeros_like(o_ref)

    o_ref[...] += contrib


def _mlp2_pool(h, q, bt3, w1, b1, w2, b2):
    return pl.pallas_call(
        _mlp2_pool_body,
        grid=(NB,),
        in_specs=[
            pl.BlockSpec((BN, H), lambda i: (i, 0)),
            pl.BlockSpec((NC, BN, H), lambda i: (0, i, 0)),
            pl.BlockSpec((1, 1, BN), lambda i: (i, 0, 0)),
            pl.BlockSpec((H, O), lambda i: (0, 0)),
            pl.BlockSpec((1, O), lambda i: (0, 0)),
            pl.BlockSpec((O, O), lambda i: (0, 0)),
            pl.BlockSpec((1, O), lambda i: (0, 0)),
        ],
        out_specs=pl.BlockSpec((G, O), lambda i: (0, 0)),
        out_shape=jax.ShapeDtypeStruct((G, O), jnp.float32),
    )(h, q, bt3, w1, b1, w2, b2)


def kernel(x, edge_index, edge_attr, batch, W1_0, b1_0, W2_0, b2_0,
           W1_1, b1_1, W2_1, b2_1):
    del edge_attr
    ei = edge_index.reshape(2 * E)
    p = _edge_aggr(x, ei)
    h1 = _mlp1(x, p, W1_0, b1_0[None], W2_0, b2_0[None])
    q = _edge_aggr(h1, ei)
    bt3 = batch.reshape(NB, 1, BN)
    return _mlp2_pool(h1, q, bt3, W1_1, b1_1[None], W2_1, b2_1[None])


# speedup vs baseline: 12.4692x; 1.0152x over previous
import functools

import jax
import jax.numpy as jnp
from jax import lax
from jax.experimental import pallas as pl
from jax.experimental.pallas import tpu as pltpu
from jax.experimental.pallas import tpu_sc as plsc

N, E, D, H, O, G = 10000, 320000, 128, 128, 128, 64
NC, NS, L = 2, 16, 16
EPW = E // (NC * NS)
C = 40
NCH = EPW // C
NBUF = 7
NW = NCH // NBUF
NTL = NCH - NBUF * NW
ZR = 16
DR = 80

BN = 1000
NB = N // BN


def _edge_aggr_body(h_hbm, ei_hbm, out_hbm, sidx_all, *scr):
    didx = scr[0:NBUF]
    rows = scr[NBUF:2 * NBUF]
    zbuf = scr[2 * NBUF]
    acc = scr[2 * NBUF + 1]
    sg = scr[2 * NBUF + 2:3 * NBUF + 2]
    ss = scr[3 * NBUF + 2:4 * NBUF + 2]
    cid = lax.axis_index("c")
    sid = lax.axis_index("s")

    ebase = (cid * NS + sid) * EPW
    pltpu.async_copy(ei_hbm.at[pl.ds(ebase, EPW)], sidx_all, ss[0])

    zv = jnp.zeros((L,), jnp.float32)

    def zb(i, _):
        zbuf[i // (D // L), pl.ds((i % (D // L)) * L, L)] = zv
        return 0

    lax.fori_loop(0, ZR * (D // L), zb, 0)

    nz = jnp.where(sid < NS - 1, 640 // ZR, 400 // ZR)
    zbase = sid * 640

    def zc(j, _):
        pltpu.sync_copy(zbuf, acc.at[pl.ds(zbase + j * ZR, ZR)])
        return 0

    lax.fori_loop(0, nz, zc, 0)

    pltpu.make_async_copy(ei_hbm.at[pl.ds(ebase, EPW)], sidx_all, ss[0]).wait()
    plsc.subcore_barrier()

    def fire_in(cn, b):
        pltpu.async_copy(ei_hbm.at[pl.ds(E + ebase + cn * C, C)], didx[b], sg[b])
        pltpu.async_copy(h_hbm.at[sidx_all.at[pl.ds(cn * C, C)]], rows[b], sg[b])

    def wait_in(cn, b):
        pltpu.make_async_copy(
            ei_hbm.at[pl.ds(E + ebase + cn * C, C)], didx[b], sg[b]).wait()
        pltpu.make_async_copy(
            h_hbm.at[sidx_all.at[pl.ds(cn * C, C)]], rows[b], sg[b]).wait()

    def fire_s(b):
        pltpu.async_copy(rows[b], acc.at[didx[b]], ss[b], add=True)

    def wait_s(b):
        pltpu.make_async_copy(rows[b], acc.at[didx[b]], ss[b]).wait()

    for b in range(NBUF):
        fire_in(b, b)

    def body(i, _):
        for b in range(NBUF):
            j = NBUF * i + b
            wait_in(j, b)
            fire_s(b)
        for b in range(NBUF):
            jn = NBUF * (i + 1) + b

            @pl.when(jn < NCH)
            def _():
                wait_s(b)
                fire_in(jn, b)
            _ = None
        return 0

    lax.fori_loop(0, NW, body, 0)
    for b in range(NTL):
        wait_in(NBUF * NW + b, b)
        fire_s(b)
    for b in range(NTL, NBUF):
        wait_s(b)
    for b in range(NTL):
        wait_s(b)

    plsc.subcore_barrier()

    nd = jnp.where(sid < NS - 1, 640 // DR, 400 // DR)

    def dc(j, _):
        b = zbase + j * DR
        pltpu.sync_copy(acc.at[pl.ds(b, DR)], out_hbm.at[cid, pl.ds(b, DR)])
        return 0

    lax.fori_loop(0, nd, dc, 0)


_edge_aggr = functools.partial(
    pl.kernel,
    out_type=jax.ShapeDtypeStruct((NC, N, D), jnp.float32),
    mesh=plsc.VectorSubcoreMesh(core_axis_name="c", subcore_axis_name="s"),
    scratch_types=(
        [pltpu.VMEM((EPW,), jnp.int32)]
        + [pltpu.VMEM((C,), jnp.int32) for _ in range(NBUF)]
        + [pltpu.VMEM((C, D), jnp.float32) for _ in range(NBUF)]
        + [pltpu.VMEM((ZR, D), jnp.float32),
           pltpu.VMEM_SHARED((N, D), jnp.float32)]
        + [pltpu.SemaphoreType.DMA for _ in range(2 * NBUF)]
    ),
)(_edge_aggr_body)


def _mlp1_body(x_ref, p_ref, w1_ref, b1_ref, w2_ref, b2_ref, o_ref):
    m = x_ref[...] + p_ref[0] + p_ref[1]
    t = jnp.dot(m, w1_ref[...], preferred_element_type=jnp.float32) + b1_ref[...]
    t = jnp.maximum(t, 0.0)
    y = jnp.dot(t, w2_ref[...], preferred_element_type=jnp.float32) + b2_ref[...]
    o_ref[...] = jnp.maximum(y, 0.0)


def _mlp1(x, p, w1, b1, w2, b2):
    return pl.pallas_call(
        _mlp1_body,
        grid=(NB,),
        in_specs=[
            pl.BlockSpec((BN, D), lambda i: (i, 0)),
            pl.BlockSpec((NC, BN, D), lambda i: (0, i, 0)),
            pl.BlockSpec((D, H), lambda i: (0, 0)),
            pl.BlockSpec((1, H), lambda i: (0, 0)),
            pl.BlockSpec((H, H), lambda i: (0, 0)),
            pl.BlockSpec((1, H), lambda i: (0, 0)),
        ],
        out_specs=pl.BlockSpec((BN, H), lambda i: (i, 0)),
        out_shape=jax.ShapeDtypeStruct((N, H), jnp.float32),
    )(x, p, w1, b1, w2, b2)


def _mlp2_pool_body(h_ref, q_ref, bt_ref, w1_ref, b1_ref, w2_ref, b2_ref, o_ref):
    m = h_ref[...] + q_ref[0] + q_ref[1]
    t = jnp.dot(m, w1_ref[...], preferred_element_type=jnp.float32) + b1_ref[...]
    t = jnp.maximum(t, 0.0)
    y = jnp.dot(t, w2_ref[...], preferred_element_type=jnp.float32) + b2_ref[...]
    bt = bt_ref[...].reshape(1, BN)
    onehot = (bt == lax.broadcasted_iota(jnp.int32, (G, BN), 0)).astype(jnp.float32)
    contrib = jnp.dot(onehot, y, preferred_element_type=jnp.float32)

    @pl.when(pl.program_id(0) == 0)
    def _():
        o_ref[...] = jnp.zeros_like(o_ref)

    o_ref[...] += contrib


def _mlp2_pool(h, q, bt3, w1, b1, w2, b2):
    return pl.pallas_call(
        _mlp2_pool_body,
        grid=(NB,),
        in_specs=[
            pl.BlockSpec((BN, H), lambda i: (i, 0)),
            pl.BlockSpec((NC, BN, H), lambda i: (0, i, 0)),
            pl.BlockSpec((1, 1, BN), lambda i: (i, 0, 0)),
            pl.BlockSpec((H, O), lambda i: (0, 0)),
            pl.BlockSpec((1, O), lambda i: (0, 0)),
            pl.BlockSpec((O, O), lambda i: (0, 0)),
            pl.BlockSpec((1, O), lambda i: (0, 0)),
        ],
        out_specs=pl.BlockSpec((G, O), lambda i: (0, 0)),
        out_shape=jax.ShapeDtypeStruct((G, O), jnp.float32),
    )(h, q, bt3, w1, b1, w2, b2)


def kernel(x, edge_index, edge_attr, batch, W1_0, b1_0, W2_0, b2_0,
           W1_1, b1_1, W2_1, b2_1):
    del edge_attr
    ei = edge_index.reshape(2 * E)
    p = _edge_aggr(x, ei)
    h1 = _mlp1(x, p, W1_0, b1_0[None], W2_0, b2_0[None])
    q = _edge_aggr(h1, ei)
    bt3 = batch.reshape(NB, 1, BN)
    return _mlp2_pool(h1, q, bt3, W1_1, b1_1[None], W2_1, b2_1[None])


# speedup vs baseline: 12.7568x; 1.0231x over previous
import functools

import jax
import jax.numpy as jnp
from jax import lax
from jax.experimental import pallas as pl
from jax.experimental.pallas import tpu as pltpu
from jax.experimental.pallas import tpu_sc as plsc

N, E, D, H, O, G = 10000, 320000, 128, 128, 128, 64
NC, NS, L = 2, 16, 16
EPW = E // (NC * NS)
C = 40
NCH = EPW // C
NBUF = 7
NW = NCH // NBUF
NTL = NCH - NBUF * NW
ZR = 16
DR = 80

BN = 2000
NB = N // BN


def _edge_aggr_body(h_hbm, ei_hbm, out_hbm, sidx_all, *scr):
    didx = scr[0:NBUF]
    rows = scr[NBUF:2 * NBUF]
    zbuf = scr[2 * NBUF]
    acc = scr[2 * NBUF + 1]
    sg = scr[2 * NBUF + 2:3 * NBUF + 2]
    ss = scr[3 * NBUF + 2:4 * NBUF + 2]
    cid = lax.axis_index("c")
    sid = lax.axis_index("s")

    ebase = (cid * NS + sid) * EPW
    pltpu.async_copy(ei_hbm.at[pl.ds(ebase, EPW)], sidx_all, ss[0])

    zv = jnp.zeros((L,), jnp.float32)

    def zb(i, _):
        zbuf[i // (D // L), pl.ds((i % (D // L)) * L, L)] = zv
        return 0

    lax.fori_loop(0, ZR * (D // L), zb, 0)

    nz = jnp.where(sid < NS - 1, 640 // ZR, 400 // ZR)
    zbase = sid * 640

    def zc(j, _):
        pltpu.sync_copy(zbuf, acc.at[pl.ds(zbase + j * ZR, ZR)])
        return 0

    lax.fori_loop(0, nz, zc, 0)

    pltpu.make_async_copy(ei_hbm.at[pl.ds(ebase, EPW)], sidx_all, ss[0]).wait()
    plsc.subcore_barrier()

    def fire_in(cn, b):
        pltpu.async_copy(h_hbm.at[sidx_all.at[pl.ds(cn * C, C)]], rows[b], sg[b])
        pltpu.async_copy(ei_hbm.at[pl.ds(E + ebase + cn * C, C)], didx[b], sg[b])

    def wait_in(cn, b):
        pltpu.make_async_copy(
            ei_hbm.at[pl.ds(E + ebase + cn * C, C)], didx[b], sg[b]).wait()
        pltpu.make_async_copy(
            h_hbm.at[sidx_all.at[pl.ds(cn * C, C)]], rows[b], sg[b]).wait()

    def fire_s(b):
        pltpu.async_copy(rows[b], acc.at[didx[b]], ss[b], add=True)

    def wait_s(b):
        pltpu.make_async_copy(rows[b], acc.at[didx[b]], ss[b]).wait()

    for b in range(NBUF):
        fire_in(b, b)

    def body(i, _):
        for b in range(NBUF):
            j = NBUF * i + b
            wait_in(j, b)
            fire_s(b)
        for b in range(NBUF):
            jn = NBUF * (i + 1) + b

            @pl.when(jn < NCH)
            def _():
                wait_s(b)
                fire_in(jn, b)
            _ = None
        return 0

    lax.fori_loop(0, NW, body, 0)
    for b in range(NTL):
        wait_in(NBUF * NW + b, b)
        fire_s(b)
    for b in range(NTL, NBUF):
        wait_s(b)
    for b in range(NTL):
        wait_s(b)

    plsc.subcore_barrier()

    nd = jnp.where(sid < NS - 1, 640 // DR, 400 // DR)

    def dc(j, _):
        b = zbase + j * DR
        pltpu.sync_copy(acc.at[pl.ds(b, DR)], out_hbm.at[cid, pl.ds(b, DR)])
        return 0

    lax.fori_loop(0, nd, dc, 0)


_edge_aggr = functools.partial(
    pl.kernel,
    out_type=jax.ShapeDtypeStruct((NC, N, D), jnp.float32),
    mesh=plsc.VectorSubcoreMesh(core_axis_name="c", subcore_axis_name="s"),
    scratch_types=(
        [pltpu.VMEM((EPW,), jnp.int32)]
        + [pltpu.VMEM((C,), jnp.int32) for _ in range(NBUF)]
        + [pltpu.VMEM((C, D), jnp.float32) for _ in range(NBUF)]
        + [pltpu.VMEM((ZR, D), jnp.float32),
           pltpu.VMEM_SHARED((N, D), jnp.float32)]
        + [pltpu.SemaphoreType.DMA for _ in range(2 * NBUF)]
    ),
)(_edge_aggr_body)


def _mlp1_body(x_ref, p_ref, w1_ref, b1_ref, w2_ref, b2_ref, o_ref):
    m = x_ref[...] + p_ref[0] + p_ref[1]
    t = jnp.dot(m, w1_ref[...], preferred_element_type=jnp.float32) + b1_ref[...]
    t = jnp.maximum(t, 0.0)
    y = jnp.dot(t, w2_ref[...], preferred_element_type=jnp.float32) + b2_ref[...]
    o_ref[...] = jnp.maximum(y, 0.0)


def _mlp1(x, p, w1, b1, w2, b2):
    return pl.pallas_call(
        _mlp1_body,
        grid=(NB,),
        in_specs=[
            pl.BlockSpec((BN, D), lambda i: (i, 0)),
            pl.BlockSpec((NC, BN, D), lambda i: (0, i, 0)),
            pl.BlockSpec((D, H), lambda i: (0, 0)),
            pl.BlockSpec((1, H), lambda i: (0, 0)),
            pl.BlockSpec((H, H), lambda i: (0, 0)),
            pl.BlockSpec((1, H), lambda i: (0, 0)),
        ],
        out_specs=pl.BlockSpec((BN, H), lambda i: (i, 0)),
        out_shape=jax.ShapeDtypeStruct((N, H), jnp.float32),
    )(x, p, w1, b1, w2, b2)


def _mlp2_pool_body(h_ref, q_ref, bt_ref, w1_ref, b1_ref, w2_ref, b2_ref, o_ref):
    m = h_ref[...] + q_ref[0] + q_ref[1]
    t = jnp.dot(m, w1_ref[...], preferred_element_type=jnp.float32) + b1_ref[...]
    t = jnp.maximum(t, 0.0)
    y = jnp.dot(t, w2_ref[...], preferred_element_type=jnp.float32) + b2_ref[...]
    bt = bt_ref[...].reshape(1, BN)
    onehot = (bt == lax.broadcasted_iota(jnp.int32, (G, BN), 0)).astype(jnp.float32)
    contrib = jnp.dot(onehot, y, preferred_element_type=jnp.float32)

    @pl.when(pl.program_id(0) == 0)
    def _():
        o_ref[...] = jnp.zeros_like(o_ref)

    o_ref[...] += contrib


def _mlp2_pool(h, q, bt3, w1, b1, w2, b2):
    return pl.pallas_call(
        _mlp2_pool_body,
        grid=(NB,),
        in_specs=[
            pl.BlockSpec((BN, H), lambda i: (i, 0)),
            pl.BlockSpec((NC, BN, H), lambda i: (0, i, 0)),
            pl.BlockSpec((1, 1, BN), lambda i: (i, 0, 0)),
            pl.BlockSpec((H, O), lambda i: (0, 0)),
            pl.BlockSpec((1, O), lambda i: (0, 0)),
            pl.BlockSpec((O, O), lambda i: (0, 0)),
            pl.BlockSpec((1, O), lambda i: (0, 0)),
        ],
        out_specs=pl.BlockSpec((G, O), lambda i: (0, 0)),
        out_shape=jax.ShapeDtypeStruct((G, O), jnp.float32),
    )(h, q, bt3, w1, b1, w2, b2)


def kernel(x, edge_index, edge_attr, batch, W1_0, b1_0, W2_0, b2_0,
           W1_1, b1_1, W2_1, b2_1):
    del edge_attr
    ei = edge_index.reshape(2 * E)
    p = _edge_aggr(x, ei)
    h1 = _mlp1(x, p, W1_0, b1_0[None], W2_0, b2_0[None])
    q = _edge_aggr(h1, ei)
    bt3 = batch.reshape(NB, 1, BN)
    return _mlp2_pool(h1, q, bt3, W1_1, b1_1[None], W2_1, b2_1[None])


# speedup vs baseline: 12.8557x; 1.0078x over previous
import functools

import jax
import jax.numpy as jnp
from jax import lax
from jax.experimental import pallas as pl
from jax.experimental.pallas import tpu as pltpu
from jax.experimental.pallas import tpu_sc as plsc

N, E, D, H, O, G = 10000, 320000, 128, 128, 128, 64
NC, NS, L = 2, 16, 16
EPW = E // (NC * NS)
C = 40
NCH = EPW // C
NBUF = 7
NW = NCH // NBUF
NTL = NCH - NBUF * NW
ZR = 16
DR = 80

BN = 5000
NB = N // BN


def _edge_aggr_body(h_hbm, ei_hbm, out_hbm, sidx_all, *scr):
    didx = scr[0:NBUF]
    rows = scr[NBUF:2 * NBUF]
    zbuf = scr[2 * NBUF]
    acc = scr[2 * NBUF + 1]
    sg = scr[2 * NBUF + 2:3 * NBUF + 2]
    ss = scr[3 * NBUF + 2:4 * NBUF + 2]
    cid = lax.axis_index("c")
    sid = lax.axis_index("s")

    ebase = (cid * NS + sid) * EPW
    pltpu.async_copy(ei_hbm.at[pl.ds(ebase, EPW)], sidx_all, ss[0])

    zv = jnp.zeros((L,), jnp.float32)

    def zb(i, _):
        zbuf[i // (D // L), pl.ds((i % (D // L)) * L, L)] = zv
        return 0

    lax.fori_loop(0, ZR * (D // L), zb, 0)

    nz = jnp.where(sid < NS - 1, 640 // ZR, 400 // ZR)
    zbase = sid * 640

    def zc(j, _):
        pltpu.sync_copy(zbuf, acc.at[pl.ds(zbase + j * ZR, ZR)])
        return 0

    lax.fori_loop(0, nz, zc, 0)

    pltpu.make_async_copy(ei_hbm.at[pl.ds(ebase, EPW)], sidx_all, ss[0]).wait()
    plsc.subcore_barrier()

    def fire_in(cn, b):
        pltpu.async_copy(h_hbm.at[sidx_all.at[pl.ds(cn * C, C)]], rows[b], sg[b])
        pltpu.async_copy(ei_hbm.at[pl.ds(E + ebase + cn * C, C)], didx[b], sg[b])

    def wait_in(cn, b):
        pltpu.make_async_copy(
            ei_hbm.at[pl.ds(E + ebase + cn * C, C)], didx[b], sg[b]).wait()
        pltpu.make_async_copy(
            h_hbm.at[sidx_all.at[pl.ds(cn * C, C)]], rows[b], sg[b]).wait()

    def fire_s(b):
        pltpu.async_copy(rows[b], acc.at[didx[b]], ss[b], add=True)

    def wait_s(b):
        pltpu.make_async_copy(rows[b], acc.at[didx[b]], ss[b]).wait()

    for b in range(NBUF):
        fire_in(b, b)

    def body(i, _):
        for b in range(NBUF):
            j = NBUF * i + b
            wait_in(j, b)
            fire_s(b)
        for b in range(NBUF):
            jn = NBUF * (i + 1) + b

            @pl.when(jn < NCH)
            def _():
                wait_s(b)
                fire_in(jn, b)
            _ = None
        return 0

    lax.fori_loop(0, NW, body, 0)
    for b in range(NTL):
        wait_in(NBUF * NW + b, b)
        fire_s(b)
    for b in range(NTL, NBUF):
        wait_s(b)
    for b in range(NTL):
        wait_s(b)

    plsc.subcore_barrier()

    nd = jnp.where(sid < NS - 1, 640 // DR, 400 // DR)

    def dc(j, _):
        b = zbase + j * DR
        pltpu.sync_copy(acc.at[pl.ds(b, DR)], out_hbm.at[cid, pl.ds(b, DR)])
        return 0

    lax.fori_loop(0, nd, dc, 0)


_edge_aggr = functools.partial(
    pl.kernel,
    out_type=jax.ShapeDtypeStruct((NC, N, D), jnp.float32),
    mesh=plsc.VectorSubcoreMesh(core_axis_name="c", subcore_axis_name="s"),
    scratch_types=(
        [pltpu.VMEM((EPW,), jnp.int32)]
        + [pltpu.VMEM((C,), jnp.int32) for _ in range(NBUF)]
        + [pltpu.VMEM((C, D), jnp.float32) for _ in range(NBUF)]
        + [pltpu.VMEM((ZR, D), jnp.float32),
           pltpu.VMEM_SHARED((N, D), jnp.float32)]
        + [pltpu.SemaphoreType.DMA for _ in range(2 * NBUF)]
    ),
)(_edge_aggr_body)


def _mlp1_body(x_ref, p_ref, w1_ref, b1_ref, w2_ref, b2_ref, o_ref):
    m = x_ref[...] + p_ref[0] + p_ref[1]
    t = jnp.dot(m, w1_ref[...], preferred_element_type=jnp.float32) + b1_ref[...]
    t = jnp.maximum(t, 0.0)
    y = jnp.dot(t, w2_ref[...], preferred_element_type=jnp.float32) + b2_ref[...]
    o_ref[...] = jnp.maximum(y, 0.0)


def _mlp1(x, p, w1, b1, w2, b2):
    return pl.pallas_call(
        _mlp1_body,
        grid=(NB,),
        in_specs=[
            pl.BlockSpec((BN, D), lambda i: (i, 0)),
            pl.BlockSpec((NC, BN, D), lambda i: (0, i, 0)),
            pl.BlockSpec((D, H), lambda i: (0, 0)),
            pl.BlockSpec((1, H), lambda i: (0, 0)),
            pl.BlockSpec((H, H), lambda i: (0, 0)),
            pl.BlockSpec((1, H), lambda i: (0, 0)),
        ],
        out_specs=pl.BlockSpec((BN, H), lambda i: (i, 0)),
        out_shape=jax.ShapeDtypeStruct((N, H), jnp.float32),
    )(x, p, w1, b1, w2, b2)


def _mlp2_pool_body(h_ref, q_ref, bt_ref, w1_ref, b1_ref, w2_ref, b2_ref, o_ref):
    m = h_ref[...] + q_ref[0] + q_ref[1]
    t = jnp.dot(m, w1_ref[...], preferred_element_type=jnp.float32) + b1_ref[...]
    t = jnp.maximum(t, 0.0)
    y = jnp.dot(t, w2_ref[...], preferred_element_type=jnp.float32) + b2_ref[...]
    bt = bt_ref[...].reshape(1, BN)
    onehot = (bt == lax.broadcasted_iota(jnp.int32, (G, BN), 0)).astype(jnp.float32)
    contrib = jnp.dot(onehot, y, preferred_element_type=jnp.float32)

    @pl.when(pl.program_id(0) == 0)
    def _():
        o_ref[...] = jnp.zeros_like(o_ref)

    o_ref[...] += contrib


def _mlp2_pool(h, q, bt3, w1, b1, w2, b2):
    return pl.pallas_call(
        _mlp2_pool_body,
        grid=(NB,),
        in_specs=[
            pl.BlockSpec((BN, H), lambda i: (i, 0)),
            pl.BlockSpec((NC, BN, H), lambda i: (0, i, 0)),
            pl.BlockSpec((1, 1, BN), lambda i: (i, 0, 0)),
            pl.BlockSpec((H, O), lambda i: (0, 0)),
            pl.BlockSpec((1, O), lambda i: (0, 0)),
            pl.BlockSpec((O, O), lambda i: (0, 0)),
            pl.BlockSpec((1, O), lambda i: (0, 0)),
        ],
        out_specs=pl.BlockSpec((G, O), lambda i: (0, 0)),
        out_shape=jax.ShapeDtypeStruct((G, O), jnp.float32),
    )(h, q, bt3, w1, b1, w2, b2)


def kernel(x, edge_index, edge_attr, batch, W1_0, b1_0, W2_0, b2_0,
           W1_1, b1_1, W2_1, b2_1):
    del edge_attr
    ei = edge_index.reshape(2 * E)
    p = _edge_aggr(x, ei)
    h1 = _mlp1(x, p, W1_0, b1_0[None], W2_0, b2_0[None])
    q = _edge_aggr(h1, ei)
    bt3 = batch.reshape(NB, 1, BN)
    return _mlp2_pool(h1, q, bt3, W1_1, b1_1[None], W2_1, b2_1[None])
